# baseline (device time: 84010 ns/iter reference)
import jax
import jax.numpy as jnp
from jax import lax
from jax.experimental import pallas as pl
from jax.experimental.pallas import tpu as pltpu

N_DEV = 4


def kernel(x, w_mat):
    m_per, k = x.shape
    _, n_per = w_mat.shape
    m_glob = N_DEV * m_per

    def body(x_ref, w_ref, out_ref, xg_ref, send_sems, recv_sems):
        my_pos = lax.axis_index("i")
        left = (my_pos - 1) % N_DEV
        right = (my_pos + 1) % N_DEV

        barrier_sem = pltpu.get_barrier_semaphore()
        pl.semaphore_signal(barrier_sem, inc=1, device_id=(left,),
                            device_id_type=pl.DeviceIdType.MESH)
        pl.semaphore_signal(barrier_sem, inc=1, device_id=(right,),
                            device_id_type=pl.DeviceIdType.MESH)
        pl.semaphore_wait(barrier_sem, 2)

        xg_ref[pl.ds(my_pos * m_per, m_per), :] = x_ref[...].astype(jnp.bfloat16)
        w = w_ref[...].astype(jnp.bfloat16)

        def gemm(origin):
            blk = xg_ref[pl.ds(origin * m_per, m_per), :]
            y = jnp.dot(blk, w, preferred_element_type=jnp.float32)
            out_ref[pl.ds(origin * m_per, m_per), :] = y * jax.nn.sigmoid(y)

        for h in range(N_DEV - 1):
            src_o = (my_pos - h) % N_DEV
            rdma = pltpu.make_async_remote_copy(
                src_ref=xg_ref.at[pl.ds(src_o * m_per, m_per), :],
                dst_ref=xg_ref.at[pl.ds(src_o * m_per, m_per), :],
                send_sem=send_sems.at[h],
                recv_sem=recv_sems.at[h],
                device_id=(right,),
                device_id_type=pl.DeviceIdType.MESH,
            )
            rdma.start()
            gemm(src_o)
            rdma.wait()

        gemm((my_pos + 1) % N_DEV)

    return pl.pallas_call(
        body,
        out_shape=jax.ShapeDtypeStruct((m_glob, n_per), jnp.float32),
        in_specs=[
            pl.BlockSpec(memory_space=pltpu.VMEM),
            pl.BlockSpec(memory_space=pltpu.VMEM),
        ],
        out_specs=pl.BlockSpec(memory_space=pltpu.VMEM),
        scratch_shapes=[
            pltpu.VMEM((m_glob, k), jnp.bfloat16),
            pltpu.SemaphoreType.DMA((N_DEV - 1,)),
            pltpu.SemaphoreType.DMA((N_DEV - 1,)),
        ],
        compiler_params=pltpu.CompilerParams(collective_id=0),
    )(x, w_mat)


# device time: 47965 ns/iter; 1.7515x vs baseline; 1.7515x over previous
import jax
import jax.numpy as jnp
from jax import lax
from jax.experimental import pallas as pl
from jax.experimental.pallas import tpu as pltpu

N_DEV = 4


def kernel(x, w_mat):
    m_per, k = x.shape
    _, n_per = w_mat.shape
    m_glob = N_DEV * m_per
    m_half = m_per // 2

    def body(x_ref, w_ref, out_ref, xg_ref, send_sems, recv_sems):
        my_pos = lax.axis_index("i")
        left = (my_pos - 1) % N_DEV
        right = (my_pos + 1) % N_DEV
        opposite = (my_pos + 2) % N_DEV

        barrier_sem = pltpu.get_barrier_semaphore()
        pl.semaphore_signal(barrier_sem, inc=1, device_id=(left,),
                            device_id_type=pl.DeviceIdType.MESH)
        pl.semaphore_signal(barrier_sem, inc=1, device_id=(right,),
                            device_id_type=pl.DeviceIdType.MESH)
        pl.semaphore_wait(barrier_sem, 2)

        xg_ref[pl.ds(my_pos * m_per, m_per), :] = x_ref[...].astype(jnp.bfloat16)

        def make_rdma(start, rows, sem, dst):
            return pltpu.make_async_remote_copy(
                src_ref=xg_ref.at[pl.ds(start, rows), :],
                dst_ref=xg_ref.at[pl.ds(start, rows), :],
                send_sem=send_sems.at[sem],
                recv_sem=recv_sems.at[sem],
                device_id=(dst,),
                device_id_type=pl.DeviceIdType.MESH,
            )

        s1r = make_rdma(my_pos * m_per, m_per, 0, right)
        s1l = make_rdma(my_pos * m_per, m_per, 1, left)
        s1r.start()
        s1l.start()

        w = w_ref[...].astype(jnp.bfloat16)

        def gemm(start, rows):
            blk = xg_ref[pl.ds(start, rows), :]
            y = jnp.dot(blk, w, preferred_element_type=jnp.float32)
            out_ref[pl.ds(start, rows), :] = y * jax.nn.sigmoid(y)

        gemm(my_pos * m_per, m_per)
        s1r.wait()
        s1l.wait()

        s2r = make_rdma(left * m_per, m_half, 2, right)
        s2l = make_rdma(right * m_per + m_half, m_half, 3, left)
        s2r.start()
        s2l.start()

        gemm(left * m_per, m_per)
        gemm(right * m_per, m_per)
        s2r.wait()
        s2l.wait()

        gemm(opposite * m_per, m_per)

    return pl.pallas_call(
        body,
        out_shape=jax.ShapeDtypeStruct((m_glob, n_per), jnp.float32),
        in_specs=[
            pl.BlockSpec(memory_space=pltpu.VMEM),
            pl.BlockSpec(memory_space=pltpu.VMEM),
        ],
        out_specs=pl.BlockSpec(memory_space=pltpu.VMEM),
        scratch_shapes=[
            pltpu.VMEM((m_glob, k), jnp.bfloat16),
            pltpu.SemaphoreType.DMA((4,)),
            pltpu.SemaphoreType.DMA((4,)),
        ],
        compiler_params=pltpu.CompilerParams(collective_id=0),
    )(x, w_mat)


# device time: 46715 ns/iter; 1.7984x vs baseline; 1.0268x over previous
import jax
import jax.numpy as jnp
from jax import lax
from jax.experimental import pallas as pl
from jax.experimental.pallas import tpu as pltpu

N_DEV = 4


def kernel(x, w_mat):
    m_per, k = x.shape
    _, n_per = w_mat.shape
    m_glob = N_DEV * m_per
    m_half = m_per // 2

    def body(x_ref, w_ref, out_ref, xg_ref, send_sems, recv_sems):
        my_pos = lax.axis_index("i")
        left = (my_pos - 1) % N_DEV
        right = (my_pos + 1) % N_DEV
        opposite = (my_pos + 2) % N_DEV

        barrier_sem = pltpu.get_barrier_semaphore()
        pl.semaphore_signal(barrier_sem, inc=1, device_id=(left,),
                            device_id_type=pl.DeviceIdType.MESH)
        pl.semaphore_signal(barrier_sem, inc=1, device_id=(right,),
                            device_id_type=pl.DeviceIdType.MESH)

        xg_ref[pl.ds(my_pos * m_per, m_half), :] = (
            x_ref[pl.ds(0, m_half), :].astype(jnp.bfloat16))
        xg_ref[pl.ds(my_pos * m_per + m_half, m_half), :] = (
            x_ref[pl.ds(m_half, m_half), :].astype(jnp.bfloat16))

        pl.semaphore_wait(barrier_sem, 2)

        def make_rdma(start, sem, dst):
            return pltpu.make_async_remote_copy(
                src_ref=xg_ref.at[pl.ds(start, m_half), :],
                dst_ref=xg_ref.at[pl.ds(start, m_half), :],
                send_sem=send_sems.at[sem],
                recv_sem=recv_sems.at[sem],
                device_id=(dst,),
                device_id_type=pl.DeviceIdType.MESH,
            )

        top = lambda o: o * m_per
        bot = lambda o: o * m_per + m_half

        r0 = make_rdma(top(my_pos), 0, right)
        r1 = make_rdma(bot(my_pos), 1, right)
        l0 = make_rdma(bot(my_pos), 3, left)
        l1 = make_rdma(top(my_pos), 4, left)
        r0.start()
        l0.start()
        r1.start()
        l1.start()

        w = w_ref[...].astype(jnp.bfloat16)

        def gemm(start, rows):
            blk = xg_ref[pl.ds(start, rows), :]
            y = jnp.dot(blk, w, preferred_element_type=jnp.float32)
            out_ref[pl.ds(start, rows), :] = y * jax.nn.sigmoid(y)

        gemm(top(my_pos), m_per)

        r0.wait_recv()
        r2 = make_rdma(top(left), 2, right)
        r2.start()
        l0.wait_recv()
        l2 = make_rdma(bot(right), 5, left)
        l2.start()

        r1.wait_recv()
        gemm(top(left), m_per)
        l1.wait_recv()
        gemm(top(right), m_per)

        r2.wait_recv()
        gemm(top(opposite), m_half)
        l2.wait_recv()
        gemm(bot(opposite), m_half)

        for rdma in (r0, r1, r2, l0, l1, l2):
            rdma.wait_send()

    return pl.pallas_call(
        body,
        out_shape=jax.ShapeDtypeStruct((m_glob, n_per), jnp.float32),
        in_specs=[
            pl.BlockSpec(memory_space=pltpu.VMEM),
            pl.BlockSpec(memory_space=pltpu.VMEM),
        ],
        out_specs=pl.BlockSpec(memory_space=pltpu.VMEM),
        scratch_shapes=[
            pltpu.VMEM((m_glob, k), jnp.bfloat16),
            pltpu.SemaphoreType.DMA((6,)),
            pltpu.SemaphoreType.DMA((6,)),
        ],
        compiler_params=pltpu.CompilerParams(collective_id=0),
    )(x, w_mat)
